# Initial kernel scaffold; baseline (speedup 1.0000x reference)
import jax
import jax.numpy as jnp
from jax.experimental import pallas as pl


def kernel(hidden_states, w_gate, w_fc1, w_fc2):
    raise NotImplementedError("write your pallas kernel here")


# speedup vs baseline: 2.1522x; 2.1522x over previous
import functools

import jax
import jax.numpy as jnp
import numpy as np
from jax.experimental import pallas as pl
from jax.experimental.pallas import tpu as pltpu

B = 1
S = 2048
T = 2048
H = 768
E = 16
F = 1024
TT = 128
NT = 32
PAD = NT * TT


def _gelu_exact(v):
    a = jnp.abs(v) * np.float32(0.7071067811865476)
    t = 1.0 / (1.0 + np.float32(0.3275911) * a)
    poly = t * (np.float32(0.254829592) + t * (np.float32(-0.284496736)
            + t * (np.float32(1.421413741) + t * (np.float32(-1.453152027)
            + t * np.float32(1.061405429)))))
    erf_a = 1.0 - poly * jnp.exp(-a * a)
    erf_v = jnp.where(v >= 0, erf_a, -erf_a)
    return np.float32(0.5) * v * (1.0 + erf_v)


def _router_body(x_ref, wg_ref, logits_ref, dest_ref, w_ref, te_ref):
    x = x_ref[...]
    wg = wg_ref[...]
    logits = jnp.dot(x, wg, preferred_element_type=jnp.float32)
    logits_ref[...] = logits
    m = jnp.max(logits, axis=1, keepdims=True)
    denom = jnp.sum(jnp.exp(logits - m), axis=1, keepdims=True)
    w_ref[...] = 1.0 / denom
    col = jax.lax.broadcasted_iota(jnp.int32, (T, E), 1)
    eid = jnp.min(jnp.where(logits == m, col, E), axis=1, keepdims=True)
    onehot = (col == eid).astype(jnp.float32)
    r = jax.lax.broadcasted_iota(jnp.int32, (TT, TT), 0)
    c = jax.lax.broadcasted_iota(jnp.int32, (TT, TT), 1)
    tril = (c <= r).astype(jnp.float32)
    blocks = []
    running = jnp.zeros((1, E), jnp.float32)
    for i in range(T // TT):
        csb = jnp.dot(tril, onehot[i * TT:(i + 1) * TT, :],
                      preferred_element_type=jnp.float32) + running
        running = csb[TT - 1:TT, :]
        blocks.append(csb)
    cs = jnp.concatenate(blocks, axis=0)
    counts = running
    padded = jnp.ceil(counts / TT) * TT
    er = jax.lax.broadcasted_iota(jnp.int32, (E, E), 0)
    ec = jax.lax.broadcasted_iota(jnp.int32, (E, E), 1)
    strict = (er < ec).astype(jnp.float32)
    base = jnp.dot(padded, strict, preferred_element_type=jnp.float32)
    pos = jnp.sum(onehot * cs, axis=1, keepdims=True) - 1.0
    tok_base = jnp.sum(onehot * base, axis=1, keepdims=True)
    dest_ref[...] = (tok_base + pos).astype(jnp.int32)
    ti = jax.lax.broadcasted_iota(jnp.int32, (NT, E), 0).astype(jnp.float32) * np.float32(TT)
    ecol = jax.lax.broadcasted_iota(jnp.int32, (NT, E), 1)
    active = jnp.logical_and(ti >= base, ti < base + padded)
    te_ref[...] = jnp.sum(jnp.where(active, ecol, 0), axis=1, keepdims=True)


def _router(x2d, w_gate):
    return pl.pallas_call(
        _router_body,
        out_shape=(
            jax.ShapeDtypeStruct((T, E), jnp.float32),
            jax.ShapeDtypeStruct((T, 1), jnp.int32),
            jax.ShapeDtypeStruct((T, 1), jnp.float32),
            jax.ShapeDtypeStruct((NT, 1), jnp.int32),
        ),
    )(x2d, w_gate)


def _dense_body(logits_ref, x_ref, w1_ref, w2_ref, out_ref):
    e = pl.program_id(0)
    logits = logits_ref[...]
    m = jnp.max(logits, axis=1, keepdims=True)
    denom = jnp.sum(jnp.exp(logits - m), axis=1, keepdims=True)
    col = jax.lax.broadcasted_iota(jnp.int32, (T, E), 1)
    eid = jnp.min(jnp.where(logits == m, col, E), axis=1, keepdims=True)
    cvec = jnp.where(eid == e, 1.0 / denom, 0.0)
    h = _gelu_exact(jnp.dot(x_ref[...], w1_ref[0], preferred_element_type=jnp.float32))
    y = jnp.dot(h, w2_ref[0], preferred_element_type=jnp.float32)

    @pl.when(e == 0)
    def _():
        out_ref[...] = jnp.zeros_like(out_ref)

    out_ref[...] += y * cvec


def _dense_experts(logits, x2d, w_fc1, w_fc2):
    return pl.pallas_call(
        _dense_body,
        grid=(E,),
        in_specs=[
            pl.BlockSpec((T, E), lambda e: (0, 0)),
            pl.BlockSpec((T, H), lambda e: (0, 0)),
            pl.BlockSpec((1, H, F), lambda e: (e, 0, 0)),
            pl.BlockSpec((1, F, H), lambda e: (e, 0, 0)),
        ],
        out_specs=pl.BlockSpec((T, H), lambda e: (0, 0)),
        out_shape=jax.ShapeDtypeStruct((T, H), jnp.float32),
    )(logits, x2d, w_fc1, w_fc2)


def kernel(hidden_states, w_gate, w_fc1, w_fc2):
    x2d = hidden_states.reshape(T, H)
    logits, dest, w, te = _router(x2d, w_gate)
    out2d = _dense_experts(logits, x2d, w_fc1, w_fc2)
    return out2d.reshape(B, S, H), logits


# speedup vs baseline: 2.2461x; 1.0436x over previous
import functools

import jax
import jax.numpy as jnp
import numpy as np
from jax import lax
from jax.experimental import pallas as pl
from jax.experimental.pallas import tpu as pltpu
from jax.experimental.pallas import tpu_sc as plsc

B = 1
S = 2048
T = 2048
H = 768
E = 16
F = 1024
TT = 128
NT = 32
PAD = NT * TT
NW = 32


def _gelu_exact(v):
    a = jnp.abs(v) * np.float32(0.7071067811865476)
    t = 1.0 / (1.0 + np.float32(0.3275911) * a)
    poly = t * (np.float32(0.254829592) + t * (np.float32(-0.284496736)
            + t * (np.float32(1.421413741) + t * (np.float32(-1.453152027)
            + t * np.float32(1.061405429)))))
    erf_a = 1.0 - poly * jnp.exp(-a * a)
    erf_v = jnp.where(v >= 0, erf_a, -erf_a)
    return np.float32(0.5) * v * (1.0 + erf_v)


def _router_body(x_ref, wg_ref, logits_ref, dest_ref, w_ref, te_ref):
    x = x_ref[...]
    wg = wg_ref[...]
    logits = jnp.dot(x, wg, preferred_element_type=jnp.float32)
    logits_ref[...] = logits
    m = jnp.max(logits, axis=1, keepdims=True)
    denom = jnp.sum(jnp.exp(logits - m), axis=1, keepdims=True)
    w_ref[...] = 1.0 / denom
    col = jax.lax.broadcasted_iota(jnp.int32, (T, E), 1)
    eid = jnp.min(jnp.where(logits == m, col, E), axis=1, keepdims=True)
    onehot = (col == eid).astype(jnp.float32)
    r = jax.lax.broadcasted_iota(jnp.int32, (TT, TT), 0)
    c = jax.lax.broadcasted_iota(jnp.int32, (TT, TT), 1)
    tril = (c <= r).astype(jnp.float32)
    blocks = []
    running = jnp.zeros((1, E), jnp.float32)
    for i in range(T // TT):
        csb = jnp.dot(tril, onehot[i * TT:(i + 1) * TT, :],
                      preferred_element_type=jnp.float32) + running
        running = csb[TT - 1:TT, :]
        blocks.append(csb)
    cs = jnp.concatenate(blocks, axis=0)
    counts = running
    padded = jnp.ceil(counts / TT) * TT
    er = jax.lax.broadcasted_iota(jnp.int32, (E, E), 0)
    ec = jax.lax.broadcasted_iota(jnp.int32, (E, E), 1)
    strict = (er < ec).astype(jnp.float32)
    base = jnp.dot(padded, strict, preferred_element_type=jnp.float32)
    pos = jnp.sum(onehot * cs, axis=1, keepdims=True) - 1.0
    tok_base = jnp.sum(onehot * base, axis=1, keepdims=True)
    dest_ref[...] = (tok_base + pos).astype(jnp.int32)
    ti = jax.lax.broadcasted_iota(jnp.int32, (NT, E), 0).astype(jnp.float32) * np.float32(TT)
    ecol = jax.lax.broadcasted_iota(jnp.int32, (NT, E), 1)
    active = jnp.logical_and(ti >= base, ti < base + padded)
    te_ref[...] = jnp.sum(jnp.where(active, ecol, 0), axis=1, keepdims=True)


def _router(x2d, w_gate):
    return pl.pallas_call(
        _router_body,
        out_shape=(
            jax.ShapeDtypeStruct((T, E), jnp.float32),
            jax.ShapeDtypeStruct((T, 1), jnp.int32),
            jax.ShapeDtypeStruct((T, 1), jnp.float32),
            jax.ShapeDtypeStruct((NT, 1), jnp.int32),
        ),
    )(x2d, w_gate)


@functools.cache
def _sc_kernels():
    mesh = plsc.VectorSubcoreMesh(core_axis_name="c", subcore_axis_name="s")

    @functools.partial(
        pl.kernel,
        out_type=(
            jax.ShapeDtypeStruct((PAD,), jnp.int32),
            jax.ShapeDtypeStruct((PAD,), jnp.float32),
        ),
        mesh=mesh,
        scratch_types=[
            pltpu.VMEM((T,), jnp.int32),
            pltpu.VMEM((T,), jnp.float32),
            pltpu.VMEM((PAD,), jnp.int32),
            pltpu.VMEM((PAD,), jnp.float32),
        ],
        compiler_params=pltpu.CompilerParams(needs_layout_passes=False),
    )
    def sc_meta(dest_hbm, w_hbm, src_hbm, ws_hbm, dest_v, w_v, src_v, ws_v):
        wid = lax.axis_index("s") * 2 + lax.axis_index("c")

        @pl.when(wid == 0)
        def _():
            pltpu.sync_copy(dest_hbm, dest_v)
            pltpu.sync_copy(w_hbm, w_v)
            zi = jnp.zeros((16,), jnp.int32)
            zf = jnp.zeros((16,), jnp.float32)

            def init(i, carry):
                src_v[pl.ds(i * 16, 16)] = zi
                ws_v[pl.ds(i * 16, 16)] = zf
                return carry

            lax.fori_loop(0, PAD // 16, init, 0)
            lane = lax.broadcasted_iota(jnp.int32, (16,), 0)

            def scat(i, carry):
                idx = dest_v[pl.ds(i * 16, 16)]
                plsc.store_scatter(src_v, [idx], lane + i * 16)
                plsc.store_scatter(ws_v, [idx], w_v[pl.ds(i * 16, 16)])
                return carry

            lax.fori_loop(0, T // 16, scat, 0)
            pltpu.sync_copy(src_v, src_hbm)
            pltpu.sync_copy(ws_v, ws_hbm)

    def make_row_gather(n_rows):
        rpt = n_rows // NW

        @functools.partial(
            pl.kernel,
            out_type=jax.ShapeDtypeStruct((n_rows, H), jnp.float32),
            mesh=mesh,
            scratch_types=[
                pltpu.VMEM((rpt,), jnp.int32),
                pltpu.VMEM((rpt, H), jnp.float32),
                pltpu.SemaphoreType.DMA,
            ],
        )
        def k(table_hbm, idx_hbm, out_hbm, idx_v, rows_v, sem):
            wid = lax.axis_index("s") * 2 + lax.axis_index("c")
            base = wid * rpt
            pltpu.sync_copy(idx_hbm.at[pl.ds(base, rpt)], idx_v)
            pltpu.async_copy(table_hbm.at[idx_v], rows_v, sem).wait()
            pltpu.sync_copy(rows_v, out_hbm.at[pl.ds(base, rpt)])

        return k

    return sc_meta, make_row_gather(PAD), make_row_gather(T)


def _group_body(te_ref, xs_ref, w1_ref, w2_ref, ws_ref, out_ref):
    h = _gelu_exact(jnp.dot(xs_ref[...], w1_ref[0],
                            preferred_element_type=jnp.float32))
    y = jnp.dot(h, w2_ref[0], preferred_element_type=jnp.float32)
    out_ref[...] = y * ws_ref[...]


def _grouped_mlp(te, xs, w_fc1, w_fc2, ws_col):
    grid_spec = pltpu.PrefetchScalarGridSpec(
        num_scalar_prefetch=1,
        grid=(NT,),
        in_specs=[
            pl.BlockSpec((TT, H), lambda i, te_r: (i, 0)),
            pl.BlockSpec((1, H, F), lambda i, te_r: (te_r[i], 0, 0)),
            pl.BlockSpec((1, F, H), lambda i, te_r: (te_r[i], 0, 0)),
            pl.BlockSpec((TT, 1), lambda i, te_r: (i, 0)),
        ],
        out_specs=pl.BlockSpec((TT, H), lambda i, te_r: (i, 0)),
    )
    return pl.pallas_call(
        _group_body,
        grid_spec=grid_spec,
        out_shape=jax.ShapeDtypeStruct((PAD, H), jnp.float32),
    )(te, xs, w_fc1, w_fc2, ws_col)


def kernel(hidden_states, w_gate, w_fc1, w_fc2):
    x2d = hidden_states.reshape(T, H)
    sc_meta, sc_gather_pad, sc_gather_tok = _sc_kernels()
    logits, dest, w, te = _router(x2d, w_gate)
    dest1 = dest.reshape(T)
    src, ws = sc_meta(dest1, w.reshape(T))
    xs = sc_gather_pad(x2d, src)
    ys = _grouped_mlp(te.reshape(NT), xs, w_fc1, w_fc2, ws.reshape(PAD, 1))
    out2d = sc_gather_tok(ys, dest1)
    return out2d.reshape(B, S, H), logits


# speedup vs baseline: 2.7928x; 1.2434x over previous
import functools

import jax
import jax.numpy as jnp
import numpy as np
from jax import lax
from jax.experimental import pallas as pl
from jax.experimental.pallas import tpu as pltpu
from jax.experimental.pallas import tpu_sc as plsc

B = 1
S = 2048
T = 2048
H = 768
E = 16
F = 1024
TT = 128
NT = 32
PAD = NT * TT
NW = 32


def _gelu_exact(v):
    a = jnp.abs(v) * np.float32(0.7071067811865476)
    t = 1.0 / (1.0 + np.float32(0.3275911) * a)
    poly = t * (np.float32(0.254829592) + t * (np.float32(-0.284496736)
            + t * (np.float32(1.421413741) + t * (np.float32(-1.453152027)
            + t * np.float32(1.061405429)))))
    erf_a = 1.0 - poly * jnp.exp(-a * a)
    erf_v = jnp.where(v >= 0, erf_a, -erf_a)
    return np.float32(0.5) * v * (1.0 + erf_v)


def _router_body(x_ref, wg_ref, logits_ref, dest_ref, w_ref, te_ref):
    x = x_ref[...]
    wg = wg_ref[...]
    logits = jnp.dot(x, wg, preferred_element_type=jnp.float32)
    logits_ref[...] = logits
    m = jnp.max(logits, axis=1, keepdims=True)
    denom = jnp.sum(jnp.exp(logits - m), axis=1, keepdims=True)
    w_ref[...] = 1.0 / denom
    col = jax.lax.broadcasted_iota(jnp.int32, (T, E), 1)
    eid = jnp.min(jnp.where(logits == m, col, E), axis=1, keepdims=True)
    onehot = (col == eid).astype(jnp.float32)
    r = jax.lax.broadcasted_iota(jnp.int32, (TT, TT), 0)
    c = jax.lax.broadcasted_iota(jnp.int32, (TT, TT), 1)
    tril = (c <= r).astype(jnp.float32)
    blocks = []
    running = jnp.zeros((1, E), jnp.float32)
    for i in range(T // TT):
        csb = jnp.dot(tril, onehot[i * TT:(i + 1) * TT, :],
                      preferred_element_type=jnp.float32) + running
        running = csb[TT - 1:TT, :]
        blocks.append(csb)
    cs = jnp.concatenate(blocks, axis=0)
    counts = running
    padded = jnp.ceil(counts / TT) * TT
    er = jax.lax.broadcasted_iota(jnp.int32, (E, E), 0)
    ec = jax.lax.broadcasted_iota(jnp.int32, (E, E), 1)
    strict = (er < ec).astype(jnp.float32)
    base = jnp.dot(padded, strict, preferred_element_type=jnp.float32)
    pos = jnp.sum(onehot * cs, axis=1, keepdims=True) - 1.0
    tok_base = jnp.sum(onehot * base, axis=1, keepdims=True)
    dest_ref[...] = (tok_base + pos).astype(jnp.int32)
    ti = jax.lax.broadcasted_iota(jnp.int32, (NT, E), 0).astype(jnp.float32) * np.float32(TT)
    ecol = jax.lax.broadcasted_iota(jnp.int32, (NT, E), 1)
    active = jnp.logical_and(ti >= base, ti < base + padded)
    te_ref[...] = jnp.sum(jnp.where(active, ecol, 0), axis=1, keepdims=True)


def _router(x2d, w_gate):
    return pl.pallas_call(
        _router_body,
        out_shape=(
            jax.ShapeDtypeStruct((T, E), jnp.float32),
            jax.ShapeDtypeStruct((T, 1), jnp.int32),
            jax.ShapeDtypeStruct((T, 1), jnp.float32),
            jax.ShapeDtypeStruct((NT, 1), jnp.int32),
        ),
    )(x2d, w_gate)


@functools.cache
def _sc_kernels():
    mesh = plsc.VectorSubcoreMesh(core_axis_name="c", subcore_axis_name="s")

    @functools.partial(
        pl.kernel,
        out_type=(
            jax.ShapeDtypeStruct((PAD,), jnp.int32),
            jax.ShapeDtypeStruct((PAD,), jnp.float32),
        ),
        mesh=mesh,
        scratch_types=[
            pltpu.VMEM((T,), jnp.int32),
            pltpu.VMEM((T,), jnp.float32),
            pltpu.VMEM((PAD,), jnp.int32),
            pltpu.VMEM((PAD,), jnp.float32),
        ],
        compiler_params=pltpu.CompilerParams(needs_layout_passes=False),
        name="sc_meta",
    )
    def sc_meta(dest_hbm, w_hbm, src_hbm, ws_hbm, dest_v, w_v, src_v, ws_v):
        wid = lax.axis_index("s") * 2 + lax.axis_index("c")

        @pl.when(wid == 0)
        def _():
            pltpu.sync_copy(dest_hbm, dest_v)
            pltpu.sync_copy(w_hbm, w_v)
            lane = lax.broadcasted_iota(jnp.int32, (16,), 0)

            @plsc.parallel_loop(0, T // 16, unroll=8)
            def _scat(i):
                idx = dest_v[pl.ds(i * 16, 16)]
                plsc.store_scatter(src_v, [idx], lane + i * 16)
                plsc.store_scatter(ws_v, [idx], w_v[pl.ds(i * 16, 16)])

            pltpu.sync_copy(src_v, src_hbm)
            pltpu.sync_copy(ws_v, ws_hbm)

    def make_row_gather(n_rows, n_table, clamp, name):
        rpt = n_rows // NW

        @functools.partial(
            pl.kernel,
            out_type=jax.ShapeDtypeStruct((n_rows, H), jnp.float32),
            mesh=mesh,
            scratch_types=[
                pltpu.VMEM((rpt,), jnp.int32),
                pltpu.VMEM((rpt, H), jnp.float32),
                pltpu.SemaphoreType.DMA,
            ],
            name=name,
        )
        def k(table_hbm, idx_hbm, out_hbm, idx_v, rows_v, sem):
            wid = lax.axis_index("s") * 2 + lax.axis_index("c")
            base = wid * rpt
            pltpu.sync_copy(idx_hbm.at[pl.ds(base, rpt)], idx_v)
            if clamp:
                for j in range(rpt // 16):
                    v = idx_v[pl.ds(j * 16, 16)]
                    idx_v[pl.ds(j * 16, 16)] = jnp.minimum(
                        jnp.maximum(v, 0), n_table - 1)
            pltpu.async_copy(table_hbm.at[idx_v], rows_v, sem).wait()
            pltpu.sync_copy(rows_v, out_hbm.at[pl.ds(base, rpt)])

        return k

    return (sc_meta,
            make_row_gather(PAD, T, True, "sc_gather_pad"),
            make_row_gather(T, PAD, False, "sc_gather_tok"))


def _group_body(te_ref, xs_ref, w1_ref, w2_ref, ws_ref, out_ref):
    h = _gelu_exact(jnp.dot(xs_ref[...], w1_ref[0],
                            preferred_element_type=jnp.float32))
    y = jnp.dot(h, w2_ref[0], preferred_element_type=jnp.float32)
    out_ref[...] = y * ws_ref[...]


def _grouped_mlp(te, xs, w_fc1, w_fc2, ws_col):
    grid_spec = pltpu.PrefetchScalarGridSpec(
        num_scalar_prefetch=1,
        grid=(NT,),
        in_specs=[
            pl.BlockSpec((TT, H), lambda i, te_r: (i, 0)),
            pl.BlockSpec((1, H, F), lambda i, te_r: (te_r[i], 0, 0)),
            pl.BlockSpec((1, F, H), lambda i, te_r: (te_r[i], 0, 0)),
            pl.BlockSpec((TT, 1), lambda i, te_r: (i, 0)),
        ],
        out_specs=pl.BlockSpec((TT, H), lambda i, te_r: (i, 0)),
    )
    return pl.pallas_call(
        _group_body,
        grid_spec=grid_spec,
        out_shape=jax.ShapeDtypeStruct((PAD, H), jnp.float32),
    )(te, xs, w_fc1, w_fc2, ws_col)


def kernel(hidden_states, w_gate, w_fc1, w_fc2):
    x2d = hidden_states.reshape(T, H)
    sc_meta, sc_gather_pad, sc_gather_tok = _sc_kernels()
    logits, dest, w, te = _router(x2d, w_gate)
    dest1 = dest.reshape(T)
    src, ws = sc_meta(dest1, w.reshape(T))
    xs = sc_gather_pad(x2d, src)
    ys = _grouped_mlp(te.reshape(NT), xs, w_fc1, w_fc2, ws.reshape(PAD, 1))
    out2d = sc_gather_tok(ys, dest1)
    return out2d.reshape(B, S, H), logits
